# baseline (speedup 1.0000x reference)
import jax
import jax.numpy as jnp
from jax import lax
from jax.experimental import pallas as pl
from jax.experimental.pallas import tpu as pltpu
from jax.experimental.pallas import tpu_sc as plsc

_B, _D, _F = 4096, 128, 16
_W = 128
_BLK = 512
_PBLK = 512
_NEG = -1e9
_NW = 32
_FSL = 3 * _B // _NW
_WSL = _B // _NW


def _proj_body(x1, x2, x3, w1, w2, w3, c1, c2, c3, mi,
               p1, p2, p3, n1, n2, n3, pp1, pp2, pp3, posb, counts):
    for x, w, c, p, n, pp in ((x1, w1, c1, p1, n1, pp1),
                              (x2, w2, c2, p2, n2, pp2),
                              (x3, w3, c3, p3, n3, pp3)):
        pr = jnp.dot(x[...], w[...],
                     preferred_element_type=jnp.float32) + c[...]
        p[...] = pr
        nv = jnp.sqrt(jnp.sum(pr * pr, axis=1, keepdims=True))
        n[...] = nv
        pp[...] = jnp.concatenate(
            [pr, nv, jnp.zeros((_PBLK, _W - _F - 1), jnp.float32)], axis=1)

    @pl.when(pl.program_id(0) == 0)
    def _pos():
        v = mi[...]
        lane = lax.broadcasted_iota(jnp.int32, (1, _B), 1)
        lane16 = lax.broadcasted_iota(jnp.int32, (1, 16), 1)
        cnt = jnp.zeros((1, 16), jnp.int32)
        pos = jnp.zeros((1, _B), jnp.int32)
        start = jnp.zeros((1, _B), jnp.int32)
        for g in (0, 1, 2, 3):
            x = jnp.where(v == g, 1, 0)
            for k in (1, 2, 4, 8, 16, 32, 64, 128, 256, 512, 1024, 2048):
                x = x + jnp.where(lane >= k, pltpu.roll(x, k, 1), 0)
            ng = jnp.broadcast_to(x[:, _B - 1:_B], (1, _B))
            pos = jnp.where(v == g, start + x - 1, pos)
            start = start + ng
            cnt = jnp.where(lane16 == g,
                            jnp.broadcast_to(x[:, _B - 1:_B], (1, 16)), cnt)
        counts[...] = cnt
        for midx in (0, 1, 2):
            posb[midx:midx + 1, :] = pos + midx * _B
        posb[3:4, :] = pos


def _sc_scatter_compact(posb, ppflat, rcflat, pos_v, rows_v, sem):
    wid = lax.axis_index("s") * 2 + lax.axis_index("c")
    base = wid * _FSL
    pltpu.sync_copy(posb.at[pl.ds(base, _FSL)], pos_v)
    pltpu.sync_copy(ppflat.at[pl.ds(base, _FSL)], rows_v)
    pltpu.async_copy(rows_v, rcflat.at[pos_v], sem).wait()


def _knn_mlp_body(cnt_ref, rowsc, bank3, nt3, mcol, wh1, ch1, wh2, ch2,
                  outp, fscr):
    b = pl.program_id(0)
    r0 = b * _BLK
    iota = lax.broadcasted_iota(jnp.int32, (_BLK, _B), 1)
    rpos = r0 + lax.broadcasted_iota(jnp.int32, (_BLK, 1), 0)
    s_m = cnt_ref[0]
    for m in range(3):
        n_m = cnt_ref[m + 1]
        rows_full = rowsc[m]
        rows = rows_full[:, 0:_F]
        active = jnp.logical_and(r0 < s_m + n_m, r0 + _BLK > s_m)

        @pl.when(active)
        def _active(m=m, s_m=s_m, n_m=n_m, rows_full=rows_full, rows=rows):
            bank_v = bank3[m]
            nr = rows_full[:, _F:_F + 1]
            simr = lax.dot_general(rows, bank_v, (((1,), (1,)), ((), ())),
                                   preferred_element_type=jnp.float32)
            denom = jnp.maximum(nr * nt3[m], 1e-8)
            avail = mcol[...] != (m + 1)
            sim = jnp.where(avail, simr / denom, _NEG)
            vals, ohs = [], []
            for k in range(3):
                mx = jnp.max(sim, axis=1, keepdims=True)
                idx = jnp.min(jnp.where(sim == mx, iota, _B),
                              axis=1, keepdims=True)
                oh = iota == idx
                vals.append(mx)
                ohs.append(oh)
                if k < 2:
                    sim = jnp.where(oh, -jnp.inf, sim)
            e1 = jnp.exp(vals[1] - vals[0])
            e2 = jnp.exp(vals[2] - vals[0])
            s = 1.0 + e1 + e2
            wm = jnp.where(ohs[0], 1.0 / s,
                           jnp.where(ohs[1], e1 / s,
                                     jnp.where(ohs[2], e2 / s, 0.0)))
            knn = jnp.dot(wm, bank_v,
                          preferred_element_type=jnp.float32)
            inrange = jnp.logical_and(rpos >= s_m, rpos < s_m + n_m)
            fscr[m] = jnp.where(inrange, knn, rows)

        @pl.when(jnp.logical_not(active))
        def _passthrough(m=m, rows=rows):
            fscr[m] = rows

        s_m = s_m + n_m

    x = jnp.concatenate([fscr[0], fscr[1], fscr[2]], axis=1)
    h = jnp.maximum(jnp.dot(x, wh1[...],
                            preferred_element_type=jnp.float32) + ch1[...],
                    0.0)
    o = jnp.dot(h, wh2[...], preferred_element_type=jnp.float32) + ch2[...]
    outp[...] = jnp.concatenate(
        [o, jnp.zeros((_BLK, _W - 1), jnp.float32)], axis=1)


def _sc_gather_restore(operm, posr, orig, pos_v, vals_v, sem):
    wid = lax.axis_index("s") * 2 + lax.axis_index("c")
    base = wid * _WSL
    pltpu.sync_copy(posr.at[pl.ds(base, _WSL)], pos_v)
    pltpu.async_copy(operm.at[pos_v], vals_v, sem).wait()
    pltpu.sync_copy(vals_v, orig.at[pl.ds(base, _WSL)])


def _full(shape):
    return pl.BlockSpec(shape, lambda *_: (0,) * len(shape))


def kernel(language, video, audio, W_language, b_language, W_video, b_video,
           W_audio, b_audio, W1, b1, W2, b2, missing_index):
    f32 = jnp.float32
    i32 = jnp.int32

    mi32 = missing_index.astype(i32)
    mirow = mi32.reshape(1, _B)
    proj_specs_in = (
        [pl.BlockSpec((_PBLK, _D), lambda b: (b, 0))] * 3
        + [_full((_D, _F))] * 3 + [_full((1, _F))] * 3
        + [_full((1, _B))])
    proj_specs_out = (
        [pl.BlockSpec((_PBLK, _F), lambda b: (b, 0))] * 3
        + [pl.BlockSpec((_PBLK, 1), lambda b: (b, 0))] * 3
        + [pl.BlockSpec((_PBLK, _W), lambda b: (b, 0))] * 3
        + [_full((4, _B)), _full((1, 16))])
    (p1, p2, p3, n1, n2, n3, pp1, pp2, pp3,
     posb, counts) = pl.pallas_call(
        _proj_body,
        grid=(_B // _PBLK,),
        in_specs=proj_specs_in,
        out_specs=proj_specs_out,
        out_shape=[jax.ShapeDtypeStruct((_B, _F), f32)] * 3
        + [jax.ShapeDtypeStruct((_B, 1), f32)] * 3
        + [jax.ShapeDtypeStruct((_B, _W), f32)] * 3
        + [jax.ShapeDtypeStruct((4, _B), i32),
           jax.ShapeDtypeStruct((1, 16), i32)],
    )(language, video, audio, W_language, W_video, W_audio,
      b_language.reshape(1, _F), b_video.reshape(1, _F),
      b_audio.reshape(1, _F), mirow)

    mesh = plsc.VectorSubcoreMesh(core_axis_name="c", subcore_axis_name="s")
    ppflat = jnp.concatenate([pp1, pp2, pp3], axis=0)
    posbf = posb.reshape(4 * _B)[0:3 * _B]
    posr = posb.reshape(4 * _B)[3 * _B:4 * _B]
    rcflat = pl.kernel(
        _sc_scatter_compact,
        out_type=jax.ShapeDtypeStruct((3 * _B, _W), f32),
        mesh=mesh,
        scratch_types=[pltpu.VMEM((_FSL,), i32),
                       pltpu.VMEM((_FSL, _W), f32),
                       pltpu.SemaphoreType.DMA],
    )(posbf, ppflat)

    bank3 = jnp.stack([p1, p2, p3])
    nt3 = jnp.stack([n1.reshape(1, _B), n2.reshape(1, _B),
                     n3.reshape(1, _B)])
    rc3s = rcflat.reshape(3, _B, _W)
    operm = pl.pallas_call(
        _knn_mlp_body,
        grid_spec=pltpu.PrefetchScalarGridSpec(
            num_scalar_prefetch=1,
            grid=(_B // _BLK,),
            in_specs=[
                pl.BlockSpec((3, _BLK, _W), lambda b, c: (0, b, 0)),
                pl.BlockSpec((3, _B, _F), lambda b, c: (0, 0, 0)),
                pl.BlockSpec((3, 1, _B), lambda b, c: (0, 0, 0)),
                pl.BlockSpec((1, _B), lambda b, c: (0, 0)),
                pl.BlockSpec((3 * _F, _F), lambda b, c: (0, 0)),
                pl.BlockSpec((1, _F), lambda b, c: (0, 0)),
                pl.BlockSpec((_F, 1), lambda b, c: (0, 0)),
                pl.BlockSpec((1, 1), lambda b, c: (0, 0)),
            ],
            out_specs=pl.BlockSpec((_BLK, _W), lambda b, c: (b, 0)),
            scratch_shapes=[pltpu.VMEM((3, _BLK, _F), f32)],
        ),
        out_shape=jax.ShapeDtypeStruct((_B, _W), f32),
    )(counts.reshape(16), rc3s, bank3, nt3, mirow,
      W1, b1.reshape(1, _F), W2, b2.reshape(1, 1))

    orig = pl.kernel(
        _sc_gather_restore,
        out_type=jax.ShapeDtypeStruct((_B, _W), f32),
        mesh=mesh,
        scratch_types=[pltpu.VMEM((_WSL,), i32),
                       pltpu.VMEM((_WSL, _W), f32),
                       pltpu.SemaphoreType.DMA],
    )(operm, posr)

    return orig[:, 0:1]


# speedup vs baseline: 1.0377x; 1.0377x over previous
import jax
import jax.numpy as jnp
from jax import lax
from jax.experimental import pallas as pl
from jax.experimental.pallas import tpu as pltpu
from jax.experimental.pallas import tpu_sc as plsc

_B, _D, _F = 4096, 128, 16
_W = 128
_BLK = 1024
_PBLK = 512
_NEG = -1e9
_NW = 32
_FSL = 3 * _B // _NW


def _proj_body(x1, x2, x3, w1, w2, w3, c1, c2, c3, mi,
               p1, p2, p3, n1, n2, n3, pp1, pp2, pp3, posb, counts):
    for x, w, c, p, n, pp in ((x1, w1, c1, p1, n1, pp1),
                              (x2, w2, c2, p2, n2, pp2),
                              (x3, w3, c3, p3, n3, pp3)):
        pr = jnp.dot(x[...], w[...],
                     preferred_element_type=jnp.float32) + c[...]
        p[...] = pr
        nv = jnp.sqrt(jnp.sum(pr * pr, axis=1, keepdims=True))
        n[...] = nv
        pp[...] = jnp.concatenate(
            [pr, nv, jnp.zeros((_PBLK, _W - _F - 1), jnp.float32)], axis=1)

    @pl.when(pl.program_id(0) == 0)
    def _pos():
        v = mi[...]
        lane = lax.broadcasted_iota(jnp.int32, (1, _B), 1)
        lane16 = lax.broadcasted_iota(jnp.int32, (1, 16), 1)
        cnt = jnp.zeros((1, 16), jnp.int32)
        for code in (1, 2, 3):
            midx = code - 1
            x = jnp.where(v == code, 1, 0)
            for k in (1, 2, 4, 8, 16, 32, 64, 128, 256, 512, 1024, 2048):
                x = x + jnp.where(lane >= k, pltpu.roll(x, k, 1), 0)
            cm = jnp.broadcast_to(x[:, _B - 1:_B], (1, _B))
            cuma = (lane + 1) - x
            pos = jnp.where(v == code, x - 1, cm + cuma - 1)
            posb[midx:midx + 1, :] = pos + midx * _B
            cnt = jnp.where(lane16 == midx,
                            jnp.broadcast_to(x[:, _B - 1:_B], (1, 16)), cnt)
        counts[...] = cnt


def _sc_scatter_compact(posb, ppflat, rcflat, pos_v, rows_v, sem):
    wid = lax.axis_index("s") * 2 + lax.axis_index("c")
    base = wid * _FSL
    pltpu.sync_copy(posb.at[pl.ds(base, _FSL)], pos_v)
    pltpu.sync_copy(ppflat.at[pl.ds(base, _FSL)], rows_v)
    pltpu.async_copy(rows_v, rcflat.at[pos_v], sem).wait()


def _knn_body(cnt_ref, rowsc, bank3, nt3, mcol, fc):
    m = pl.program_id(0)
    b = pl.program_id(1)
    count = cnt_ref[m]
    r0 = b * _BLK
    rows_full = rowsc[0]
    rows = rows_full[:, 0:_F]

    @pl.when(r0 < count)
    def _active():
        bank_v = bank3[0]
        nr = rows_full[:, _F:_F + 1]
        simr = lax.dot_general(rows, bank_v, (((1,), (1,)), ((), ())),
                               preferred_element_type=jnp.float32)
        denom = jnp.maximum(nr * nt3[0], 1e-8)
        avail = mcol[...] != (m + 1)
        sim = jnp.where(avail, simr / denom, _NEG)
        iota = lax.broadcasted_iota(jnp.int32, (_BLK, _B), 1)
        vals, ohs = [], []
        for k in range(3):
            mx = jnp.max(sim, axis=1, keepdims=True)
            idx = jnp.min(jnp.where(sim == mx, iota, _B),
                          axis=1, keepdims=True)
            oh = iota == idx
            vals.append(mx)
            ohs.append(oh)
            if k < 2:
                sim = jnp.where(oh, -jnp.inf, sim)
        e1 = jnp.exp(vals[1] - vals[0])
        e2 = jnp.exp(vals[2] - vals[0])
        s = 1.0 + e1 + e2
        wm = jnp.where(ohs[0], 1.0 / s,
                       jnp.where(ohs[1], e1 / s,
                                 jnp.where(ohs[2], e2 / s, 0.0)))
        knn = jnp.dot(wm, bank_v,
                      preferred_element_type=jnp.float32)
        rpos = r0 + lax.broadcasted_iota(jnp.int32, (_BLK, 1), 0)
        fill = jnp.where(rpos < count, knn, rows)
        fc[0] = jnp.concatenate([fill, rows_full[:, _F:]], axis=1)

    @pl.when(r0 >= count)
    def _passthrough():
        fc[0] = rows_full


def _sc_gather_restore(fcflat, posb, pfflat, pos_v, vals_v, sem):
    wid = lax.axis_index("s") * 2 + lax.axis_index("c")
    base = wid * _FSL
    pltpu.sync_copy(posb.at[pl.ds(base, _FSL)], pos_v)
    pltpu.async_copy(fcflat.at[pos_v], vals_v, sem).wait()
    pltpu.sync_copy(vals_v, pfflat.at[pl.ds(base, _FSL)])


def _mlp_body(f1, f2, f3, w1, c1, w2, c2, out):
    x = jnp.concatenate([f1[...][:, 0:_F], f2[...][:, 0:_F],
                         f3[...][:, 0:_F]], axis=1)
    h = jnp.maximum(jnp.dot(x, w1[...],
                            preferred_element_type=jnp.float32) + c1[...], 0.0)
    out[...] = jnp.dot(h, w2[...],
                       preferred_element_type=jnp.float32) + c2[...]


def _full(shape):
    return pl.BlockSpec(shape, lambda *_: (0,) * len(shape))


def kernel(language, video, audio, W_language, b_language, W_video, b_video,
           W_audio, b_audio, W1, b1, W2, b2, missing_index):
    f32 = jnp.float32
    i32 = jnp.int32

    mi32 = missing_index.astype(i32)
    mirow = mi32.reshape(1, _B)
    proj_specs_in = (
        [pl.BlockSpec((_PBLK, _D), lambda b: (b, 0))] * 3
        + [_full((_D, _F))] * 3 + [_full((1, _F))] * 3
        + [_full((1, _B))])
    proj_specs_out = (
        [pl.BlockSpec((_PBLK, _F), lambda b: (b, 0))] * 3
        + [pl.BlockSpec((_PBLK, 1), lambda b: (b, 0))] * 3
        + [pl.BlockSpec((_PBLK, _W), lambda b: (b, 0))] * 3
        + [_full((3, _B)), _full((1, 16))])
    (p1, p2, p3, n1, n2, n3, pp1, pp2, pp3,
     posb, counts) = pl.pallas_call(
        _proj_body,
        grid=(_B // _PBLK,),
        in_specs=proj_specs_in,
        out_specs=proj_specs_out,
        out_shape=[jax.ShapeDtypeStruct((_B, _F), f32)] * 3
        + [jax.ShapeDtypeStruct((_B, 1), f32)] * 3
        + [jax.ShapeDtypeStruct((_B, _W), f32)] * 3
        + [jax.ShapeDtypeStruct((3, _B), i32),
           jax.ShapeDtypeStruct((1, 16), i32)],
    )(language, video, audio, W_language, W_video, W_audio,
      b_language.reshape(1, _F), b_video.reshape(1, _F),
      b_audio.reshape(1, _F), mirow)

    mesh = plsc.VectorSubcoreMesh(core_axis_name="c", subcore_axis_name="s")
    ppflat = jnp.concatenate([pp1, pp2, pp3], axis=0)
    posbf = posb.reshape(3 * _B)
    rcflat = pl.kernel(
        _sc_scatter_compact,
        out_type=jax.ShapeDtypeStruct((3 * _B, _W), f32),
        mesh=mesh,
        scratch_types=[pltpu.VMEM((_FSL,), i32),
                       pltpu.VMEM((_FSL, _W), f32),
                       pltpu.SemaphoreType.DMA],
    )(posbf, ppflat)

    bank3 = jnp.stack([p1, p2, p3])
    nt3 = jnp.stack([n1.reshape(1, _B), n2.reshape(1, _B),
                     n3.reshape(1, _B)])
    rc3s = rcflat.reshape(3, _B, _W)
    fc = pl.pallas_call(
        _knn_body,
        grid_spec=pltpu.PrefetchScalarGridSpec(
            num_scalar_prefetch=1,
            grid=(3, _B // _BLK),
            in_specs=[
                pl.BlockSpec((1, _BLK, _W), lambda m, b, c: (m, b, 0)),
                pl.BlockSpec((1, _B, _F), lambda m, b, c: (m, 0, 0)),
                pl.BlockSpec((1, 1, _B), lambda m, b, c: (m, 0, 0)),
                pl.BlockSpec((1, _B), lambda m, b, c: (0, 0)),
            ],
            out_specs=pl.BlockSpec((1, _BLK, _W), lambda m, b, c: (m, b, 0)),
        ),
        out_shape=jax.ShapeDtypeStruct((3, _B, _W), f32),
    )(counts.reshape(16), rc3s, bank3, nt3, mirow)

    pfflat = pl.kernel(
        _sc_gather_restore,
        out_type=jax.ShapeDtypeStruct((3 * _B, _W), f32),
        mesh=mesh,
        scratch_types=[pltpu.VMEM((_FSL,), i32),
                       pltpu.VMEM((_FSL, _W), f32),
                       pltpu.SemaphoreType.DMA],
    )(fc.reshape(3 * _B, _W), posbf)
    pf1 = pfflat[0:_B]
    pf2 = pfflat[_B:2 * _B]
    pf3 = pfflat[2 * _B:3 * _B]

    out = pl.pallas_call(
        _mlp_body,
        grid=(_B // _PBLK,),
        in_specs=[pl.BlockSpec((_PBLK, _W), lambda b: (b, 0))] * 3
        + [_full((3 * _F, _F)), _full((1, _F)), _full((_F, 1)),
           _full((1, 1))],
        out_specs=pl.BlockSpec((_PBLK, 1), lambda b: (b, 0)),
        out_shape=jax.ShapeDtypeStruct((_B, 1), f32),
    )(pf1, pf2, pf3, W1, b1.reshape(1, _F), W2, b2.reshape(1, 1))
    return out


# speedup vs baseline: 1.0674x; 1.0287x over previous
import jax
import jax.numpy as jnp
from jax import lax
from jax.experimental import pallas as pl
from jax.experimental.pallas import tpu as pltpu
from jax.experimental.pallas import tpu_sc as plsc

_B, _D, _F = 4096, 128, 16
_W = 128
_BLK = 512
_PBLK = 512
_NEG = -1e9
_NW = 32
_FSL = 3 * _B // _NW


def _proj_body(x1, x2, x3, w1, w2, w3, c1, c2, c3, mi,
               p1, p2, p3, n1, n2, n3, pp1, pp2, pp3, posb, counts):
    for x, w, c, p, n, pp in ((x1, w1, c1, p1, n1, pp1),
                              (x2, w2, c2, p2, n2, pp2),
                              (x3, w3, c3, p3, n3, pp3)):
        pr = jnp.dot(x[...], w[...],
                     preferred_element_type=jnp.float32) + c[...]
        p[...] = pr
        nv = jnp.sqrt(jnp.sum(pr * pr, axis=1, keepdims=True))
        n[...] = nv
        pp[...] = jnp.concatenate(
            [pr, nv, jnp.zeros((_PBLK, _W - _F - 1), jnp.float32)], axis=1)

    @pl.when(pl.program_id(0) == 0)
    def _pos():
        v = mi[...]
        lane = lax.broadcasted_iota(jnp.int32, (1, _B), 1)
        lane16 = lax.broadcasted_iota(jnp.int32, (1, 16), 1)
        cnt = jnp.zeros((1, 16), jnp.int32)
        for code in (1, 2, 3):
            midx = code - 1
            x = jnp.where(v == code, 1, 0)
            for k in (1, 2, 4, 8, 16, 32, 64, 128, 256, 512, 1024, 2048):
                x = x + jnp.where(lane >= k, pltpu.roll(x, k, 1), 0)
            cm = jnp.broadcast_to(x[:, _B - 1:_B], (1, _B))
            cuma = (lane + 1) - x
            pos = jnp.where(v == code, x - 1, cm + cuma - 1)
            posb[midx:midx + 1, :] = pos + midx * _B
            cnt = jnp.where(lane16 == midx,
                            jnp.broadcast_to(x[:, _B - 1:_B], (1, 16)), cnt)
        counts[...] = cnt


def _sc_scatter_compact(posb, ppflat, rcflat, pos_v, rows_v, sem):
    wid = lax.axis_index("s") * 2 + lax.axis_index("c")
    base = wid * _FSL
    pltpu.sync_copy(posb.at[pl.ds(base, _FSL)], pos_v)
    pltpu.sync_copy(ppflat.at[pl.ds(base, _FSL)], rows_v)
    pltpu.async_copy(rows_v, rcflat.at[pos_v], sem).wait()


def _knn_body(cnt_ref, rowsc, bank3, nt3, mcol, fc):
    m = pl.program_id(0)
    b = pl.program_id(1)
    count = cnt_ref[m]
    r0 = b * _BLK
    rows_full = rowsc[0]
    rows = rows_full[:, 0:_F]

    @pl.when(r0 < count)
    def _active():
        bank_v = bank3[0]
        nr = rows_full[:, _F:_F + 1]
        simr = lax.dot_general(rows, bank_v, (((1,), (1,)), ((), ())),
                               preferred_element_type=jnp.float32)
        denom = jnp.maximum(nr * nt3[0], 1e-8)
        avail = mcol[...] != (m + 1)
        sim = jnp.where(avail, simr / denom, _NEG)
        iota = lax.broadcasted_iota(jnp.int32, (_BLK, _B), 1)
        vals, ohs = [], []
        for k in range(3):
            mx = jnp.max(sim, axis=1, keepdims=True)
            idx = jnp.min(jnp.where(sim == mx, iota, _B),
                          axis=1, keepdims=True)
            oh = iota == idx
            vals.append(mx)
            ohs.append(oh)
            if k < 2:
                sim = jnp.where(oh, -jnp.inf, sim)
        e1 = jnp.exp(vals[1] - vals[0])
        e2 = jnp.exp(vals[2] - vals[0])
        s = 1.0 + e1 + e2
        wm = jnp.where(ohs[0], 1.0 / s,
                       jnp.where(ohs[1], e1 / s,
                                 jnp.where(ohs[2], e2 / s, 0.0)))
        knn = jnp.dot(wm, bank_v,
                      preferred_element_type=jnp.float32)
        rpos = r0 + lax.broadcasted_iota(jnp.int32, (_BLK, 1), 0)
        fill = jnp.where(rpos < count, knn, rows)
        fc[0] = jnp.concatenate([fill, rows_full[:, _F:]], axis=1)

    @pl.when(r0 >= count)
    def _passthrough():
        fc[0] = rows_full


def _sc_gather_restore(fcflat, posb, pfflat, pos_v, vals_v, sem):
    wid = lax.axis_index("s") * 2 + lax.axis_index("c")
    base = wid * _FSL
    pltpu.sync_copy(posb.at[pl.ds(base, _FSL)], pos_v)
    pltpu.async_copy(fcflat.at[pos_v], vals_v, sem).wait()
    pltpu.sync_copy(vals_v, pfflat.at[pl.ds(base, _FSL)])


def _mlp_body(f1, f2, f3, w1, c1, w2, c2, out):
    x = jnp.concatenate([f1[...][:, 0:_F], f2[...][:, 0:_F],
                         f3[...][:, 0:_F]], axis=1)
    h = jnp.maximum(jnp.dot(x, w1[...],
                            preferred_element_type=jnp.float32) + c1[...], 0.0)
    out[...] = jnp.dot(h, w2[...],
                       preferred_element_type=jnp.float32) + c2[...]


def _full(shape):
    return pl.BlockSpec(shape, lambda *_: (0,) * len(shape))


def kernel(language, video, audio, W_language, b_language, W_video, b_video,
           W_audio, b_audio, W1, b1, W2, b2, missing_index):
    f32 = jnp.float32
    i32 = jnp.int32

    mi32 = missing_index.astype(i32)
    mirow = mi32.reshape(1, _B)
    proj_specs_in = (
        [pl.BlockSpec((_PBLK, _D), lambda b: (b, 0))] * 3
        + [_full((_D, _F))] * 3 + [_full((1, _F))] * 3
        + [_full((1, _B))])
    proj_specs_out = (
        [pl.BlockSpec((_PBLK, _F), lambda b: (b, 0))] * 3
        + [pl.BlockSpec((_PBLK, 1), lambda b: (b, 0))] * 3
        + [pl.BlockSpec((_PBLK, _W), lambda b: (b, 0))] * 3
        + [_full((3, _B)), _full((1, 16))])
    (p1, p2, p3, n1, n2, n3, pp1, pp2, pp3,
     posb, counts) = pl.pallas_call(
        _proj_body,
        grid=(_B // _PBLK,),
        in_specs=proj_specs_in,
        out_specs=proj_specs_out,
        out_shape=[jax.ShapeDtypeStruct((_B, _F), f32)] * 3
        + [jax.ShapeDtypeStruct((_B, 1), f32)] * 3
        + [jax.ShapeDtypeStruct((_B, _W), f32)] * 3
        + [jax.ShapeDtypeStruct((3, _B), i32),
           jax.ShapeDtypeStruct((1, 16), i32)],
    )(language, video, audio, W_language, W_video, W_audio,
      b_language.reshape(1, _F), b_video.reshape(1, _F),
      b_audio.reshape(1, _F), mirow)

    mesh = plsc.VectorSubcoreMesh(core_axis_name="c", subcore_axis_name="s")
    ppflat = jnp.concatenate([pp1, pp2, pp3], axis=0)
    posbf = posb.reshape(3 * _B)
    rcflat = pl.kernel(
        _sc_scatter_compact,
        out_type=jax.ShapeDtypeStruct((3 * _B, _W), f32),
        mesh=mesh,
        scratch_types=[pltpu.VMEM((_FSL,), i32),
                       pltpu.VMEM((_FSL, _W), f32),
                       pltpu.SemaphoreType.DMA],
    )(posbf, ppflat)

    bank3 = jnp.stack([p1, p2, p3])
    nt3 = jnp.stack([n1.reshape(1, _B), n2.reshape(1, _B),
                     n3.reshape(1, _B)])
    rc3s = rcflat.reshape(3, _B, _W)
    fc = pl.pallas_call(
        _knn_body,
        grid_spec=pltpu.PrefetchScalarGridSpec(
            num_scalar_prefetch=1,
            grid=(3, _B // _BLK),
            in_specs=[
                pl.BlockSpec((1, _BLK, _W), lambda m, b, c: (m, b, 0)),
                pl.BlockSpec((1, _B, _F), lambda m, b, c: (m, 0, 0)),
                pl.BlockSpec((1, 1, _B), lambda m, b, c: (m, 0, 0)),
                pl.BlockSpec((1, _B), lambda m, b, c: (0, 0)),
            ],
            out_specs=pl.BlockSpec((1, _BLK, _W), lambda m, b, c: (m, b, 0)),
        ),
        out_shape=jax.ShapeDtypeStruct((3, _B, _W), f32),
    )(counts.reshape(16), rc3s, bank3, nt3, mirow)

    pfflat = pl.kernel(
        _sc_gather_restore,
        out_type=jax.ShapeDtypeStruct((3 * _B, _W), f32),
        mesh=mesh,
        scratch_types=[pltpu.VMEM((_FSL,), i32),
                       pltpu.VMEM((_FSL, _W), f32),
                       pltpu.SemaphoreType.DMA],
    )(fc.reshape(3 * _B, _W), posbf)
    pf1 = pfflat[0:_B]
    pf2 = pfflat[_B:2 * _B]
    pf3 = pfflat[2 * _B:3 * _B]

    out = pl.pallas_call(
        _mlp_body,
        grid=(_B // _PBLK,),
        in_specs=[pl.BlockSpec((_PBLK, _W), lambda b: (b, 0))] * 3
        + [_full((3 * _F, _F)), _full((1, _F)), _full((_F, 1)),
           _full((1, 1))],
        out_specs=pl.BlockSpec((_PBLK, 1), lambda b: (b, 0)),
        out_shape=jax.ShapeDtypeStruct((_B, 1), f32),
    )(pf1, pf2, pf3, W1, b1.reshape(1, _F), W2, b2.reshape(1, 1))
    return out
